# baseline (speedup 1.0000x reference)
import functools

import jax
import jax.numpy as jnp
from jax import lax
from jax.experimental import pallas as pl
from jax.experimental.pallas import tpu as pltpu
from jax.experimental.pallas import tpu_sc as plsc

_NUM_CLASSES = 64
_EPS_WITHIN = 1e-4
_EPS_BETWEEN = 1e-4

_D = 128
_N = 16 * 4096
_NT = 3 * _N // 4
_SC_ROWS = _N - _NT
_NC = 2
_NW = 16 * _NC
_ROWS_PER_W = _SC_ROWS // _NW
_CHUNK = 256
_NCHUNKS = _ROWS_PER_W // _CHUNK
_TB = 8192
_AW = 160
_ACC_WORDS = _NUM_CLASSES * _AW


def _sc_accumulate(z2, c2):
    mesh = plsc.VectorSubcoreMesh(
        core_axis_name="c", subcore_axis_name="s", num_cores=_NC)

    @functools.partial(
        pl.kernel,
        out_type=jax.ShapeDtypeStruct((_NW, _ACC_WORDS), jnp.float32),
        mesh=mesh,
        scratch_types=[
            pltpu.VMEM((_CHUNK, _D), jnp.float32),
            pltpu.VMEM((_CHUNK, _D), jnp.float32),
            pltpu.VMEM((_ROWS_PER_W,), jnp.int32),
            pltpu.VMEM((_ACC_WORDS,), jnp.float32),
            pltpu.SemaphoreType.DMA,
            pltpu.SemaphoreType.DMA,
        ],
    )
    def body(z_hbm, c_hbm, out_hbm, zbuf0, zbuf1, cbuf, acc, sem0, sem1):
        wid = lax.axis_index("s") * _NC + lax.axis_index("c")
        base = _NT + wid * _ROWS_PER_W
        zbufs = (zbuf0, zbuf1)
        sems = (sem0, sem1)

        def zero_body(i, carry):
            acc[pl.ds(i * 16, 16)] = jnp.zeros((16,), jnp.float32)
            return carry

        copies = [None] * _NCHUNKS

        def start(cc):
            copies[cc] = pltpu.async_copy(
                z_hbm.at[pl.ds(base + cc * _CHUNK, _CHUNK), :],
                zbufs[cc % 2], sems[cc % 2])

        start(0)
        pltpu.sync_copy(c_hbm.at[pl.ds(base, _ROWS_PER_W)], cbuf)
        lax.fori_loop(0, _ACC_WORDS // 16, zero_body, None)

        ones = jnp.ones((16,), jnp.float32)
        zvec = jnp.zeros((16,), jnp.float32)

        for cc in range(_NCHUNKS):
            if cc + 1 < _NCHUNKS:
                start(cc + 1)
            copies[cc].wait()
            zbuf = zbufs[cc % 2]

            def grp_body(g, carry, zbuf=zbuf, cc=cc):
                cvec = cbuf[pl.ds(cc * _CHUNK + g * 16, 16)]
                cls0 = cvec[0]
                uniform = cls0 == cvec[15]

                def fast():
                    off = cls0 * _AW
                    s = [zvec] * (_D // 16)
                    sq = [zvec] * (_D // 16)
                    for t in range(16):
                        r = g * 16 + t
                        for j in range(_D // 16):
                            v = zbuf[r, pl.ds(16 * j, 16)]
                            s[j] = s[j] + v
                            sq[j] = v * v + sq[j]
                    for j in range(_D // 16):
                        plsc.addupdate(acc.at[pl.ds(off + 16 * j, 16)], s[j])
                    sqt = sq[0]
                    for j in range(1, _D // 16):
                        sqt = sqt + sq[j]
                    plsc.addupdate(acc.at[pl.ds(off + _D, 16)], sqt)
                    plsc.addupdate(acc.at[pl.ds(off + _D + 16, 16)], 16.0 * ones)

                def slow():
                    for t in range(16):
                        cls = cvec[t]
                        off = cls * _AW
                        r = g * 16 + t
                        sq = zvec
                        for j in range(_D // 16):
                            v = zbuf[r, pl.ds(16 * j, 16)]
                            plsc.addupdate(acc.at[pl.ds(off + 16 * j, 16)], v)
                            sq = v * v + sq
                        plsc.addupdate(acc.at[pl.ds(off + _D, 16)], sq)
                        plsc.addupdate(acc.at[pl.ds(off + _D + 16, 16)], ones)

                lax.cond(uniform, fast, slow)
                return carry

            lax.fori_loop(0, _CHUNK // 16, grp_body, None)

        pltpu.sync_copy(acc, out_hbm.at[wid])

    return body(z2, c2)


def _tc_partial_finalize(z2, c2, partials_sc):
    def body(z_ref, c_ref, p_ref, o_ref, acc):
        @pl.when(pl.program_id(0) == 0)
        def _():
            acc[...] = jnp.zeros_like(acc)

        zb = z_ref[...]
        cb = c_ref[...]
        onehot_t = (
            lax.broadcasted_iota(jnp.int32, (_NUM_CLASSES, _TB), 0)
            == cb[None, :]
        ).astype(jnp.float32)
        s1 = jnp.dot(onehot_t, zb, preferred_element_type=jnp.float32)
        norms = jnp.sum(zb * zb, axis=1)
        ssq = jnp.sum(onehot_t * norms[None, :], axis=1, keepdims=True)
        cnt = jnp.sum(onehot_t, axis=1, keepdims=True)
        ssq_pad = jnp.pad(ssq, ((0, 0), (0, 15)))
        cnt_pad = jnp.pad(cnt, ((0, 0), (0, 15)))
        acc[...] += jnp.concatenate([s1, ssq_pad, cnt_pad], axis=1)

        @pl.when(pl.program_id(0) == _NT // _TB - 1)
        def _():
            a = jnp.sum(p_ref[...], axis=0) + acc[...]
            sums = a[:, :_D]
            ssq_k = jnp.sum(a[:, _D:_D + 16], axis=1)
            cnt_k = a[:, _D + 16]
            safe = jnp.maximum(cnt_k, 1.0)
            s2 = jnp.sum(sums * sums, axis=1)
            var_w = (ssq_k - s2 / safe) / safe
            total = jnp.sum(cnt_k)
            g = jnp.sum(sums, axis=0) / total
            var_b = jnp.sum(s2 / safe) / total - jnp.sum(g * g)
            loss = (jnp.mean(var_w) + _EPS_WITHIN) / (var_b + _EPS_BETWEEN)
            o_ref[0, 0] = loss

    return pl.pallas_call(
        body,
        grid=(_NT // _TB,),
        in_specs=[
            pl.BlockSpec((_TB, _D), lambda i: (i, 0)),
            pl.BlockSpec((_TB,), lambda i: (i,)),
            pl.BlockSpec((_NW, _NUM_CLASSES, _AW), lambda i: (0, 0, 0)),
        ],
        out_specs=pl.BlockSpec(
            (1, 1), lambda i: (0, 0), memory_space=pltpu.SMEM),
        out_shape=jax.ShapeDtypeStruct((1, 1), jnp.float32),
        scratch_shapes=[pltpu.VMEM((_NUM_CLASSES, _AW), jnp.float32)],
        compiler_params=pltpu.CompilerParams(
            dimension_semantics=("arbitrary",)),
    )(z2, c2, partials_sc)


@jax.jit
def kernel(z, c):
    z2 = z.reshape(-1, _D)
    c2 = c.reshape(-1)
    partials_sc = _sc_accumulate(z2, c2)
    return _tc_partial_finalize(
        z2, c2, partials_sc.reshape(_NW, _NUM_CLASSES, _AW))[0, 0]


# speedup vs baseline: 1.2660x; 1.2660x over previous
import functools

import jax
import jax.numpy as jnp
from jax import lax
from jax.experimental import pallas as pl
from jax.experimental.pallas import tpu as pltpu
from jax.experimental.pallas import tpu_sc as plsc

_NUM_CLASSES = 64
_EPS_WITHIN = 1e-4
_EPS_BETWEEN = 1e-4

_D = 128
_N = 16 * 4096
_NT = 3 * _N // 4
_SC_ROWS = _N - _NT
_NC = 2
_NW = 16 * _NC
_ROWS_PER_W = _SC_ROWS // _NW
_CHUNK = 256
_NCHUNKS = _ROWS_PER_W // _CHUNK
_TB = 8192
_AW = 160
_ACC_WORDS = _NUM_CLASSES * _AW


def _sc_accumulate(z2, c2):
    mesh = plsc.VectorSubcoreMesh(
        core_axis_name="c", subcore_axis_name="s", num_cores=_NC)

    @functools.partial(
        pl.kernel,
        out_type=jax.ShapeDtypeStruct((_NW, _ACC_WORDS), jnp.float32),
        mesh=mesh,
        scratch_types=[
            pltpu.VMEM((_CHUNK, _D), jnp.float32),
            pltpu.VMEM((_CHUNK, _D), jnp.float32),
            pltpu.VMEM((_ROWS_PER_W,), jnp.int32),
            pltpu.VMEM((_ACC_WORDS,), jnp.float32),
            pltpu.SemaphoreType.DMA,
            pltpu.SemaphoreType.DMA,
        ],
    )
    def body(z_hbm, c_hbm, out_hbm, zbuf0, zbuf1, cbuf, acc, sem0, sem1):
        wid = lax.axis_index("s") * _NC + lax.axis_index("c")
        base = _NT + wid * _ROWS_PER_W
        zbufs = (zbuf0, zbuf1)
        sems = (sem0, sem1)

        def zero_body(i, carry):
            acc[pl.ds(i * 16, 16)] = jnp.zeros((16,), jnp.float32)
            return carry

        copies = [None] * _NCHUNKS

        def start(cc):
            copies[cc] = pltpu.async_copy(
                z_hbm.at[pl.ds(base + cc * _CHUNK, _CHUNK), :],
                zbufs[cc % 2], sems[cc % 2])

        start(0)
        pltpu.sync_copy(c_hbm.at[pl.ds(base, _ROWS_PER_W)], cbuf)
        lax.fori_loop(0, _ACC_WORDS // 16, zero_body, None)

        ones = jnp.ones((16,), jnp.float32)
        zvec = jnp.zeros((16,), jnp.float32)

        for cc in range(_NCHUNKS):
            if cc + 1 < _NCHUNKS:
                start(cc + 1)
            copies[cc].wait()
            zbuf = zbufs[cc % 2]

            def grp_body(g, carry, zbuf=zbuf, cc=cc):
                cvec = cbuf[pl.ds(cc * _CHUNK + g * 16, 16)]
                cls0 = cvec[0]
                uniform = cls0 == cvec[15]

                def fast():
                    off = cls0 * _AW
                    s = [zvec] * (_D // 16)
                    sq = [zvec] * (_D // 16)
                    for t in range(16):
                        r = g * 16 + t
                        for j in range(_D // 16):
                            v = zbuf[r, pl.ds(16 * j, 16)]
                            s[j] = s[j] + v
                            sq[j] = v * v + sq[j]
                    for j in range(_D // 16):
                        plsc.addupdate(acc.at[pl.ds(off + 16 * j, 16)], s[j])
                    sqt = sq[0]
                    for j in range(1, _D // 16):
                        sqt = sqt + sq[j]
                    plsc.addupdate(acc.at[pl.ds(off + _D, 16)], sqt)
                    plsc.addupdate(acc.at[pl.ds(off + _D + 16, 16)], 16.0 * ones)

                def slow():
                    for t in range(16):
                        cls = cvec[t]
                        off = cls * _AW
                        r = g * 16 + t
                        sq = zvec
                        for j in range(_D // 16):
                            v = zbuf[r, pl.ds(16 * j, 16)]
                            plsc.addupdate(acc.at[pl.ds(off + 16 * j, 16)], v)
                            sq = v * v + sq
                        plsc.addupdate(acc.at[pl.ds(off + _D, 16)], sq)
                        plsc.addupdate(acc.at[pl.ds(off + _D + 16, 16)], ones)

                lax.cond(uniform, fast, slow)
                return carry

            lax.fori_loop(0, _CHUNK // 16, grp_body, None)

        pltpu.sync_copy(acc, out_hbm.at[wid])

    return body(z2, c2)


def _tc_partial(z2, c2):
    def body(z_ref, c_ref, o_ref):
        @pl.when(pl.program_id(0) == 0)
        def _():
            o_ref[...] = jnp.zeros_like(o_ref)

        zb = z_ref[...]
        cb = c_ref[...]
        onehot_t = (
            lax.broadcasted_iota(jnp.int32, (_NUM_CLASSES, _TB), 0)
            == cb[None, :]
        ).astype(jnp.float32)
        s1 = jnp.dot(onehot_t, zb, preferred_element_type=jnp.float32)
        sq = jnp.dot(onehot_t, zb * zb,
                     preferred_element_type=jnp.float32)
        ssq = jnp.sum(sq, axis=1, keepdims=True)
        cnt = jnp.sum(onehot_t, axis=1, keepdims=True)
        ssq_pad = jnp.pad(ssq, ((0, 0), (0, 15)))
        cnt_pad = jnp.pad(cnt, ((0, 0), (0, 15)))
        o_ref[...] += jnp.concatenate([s1, ssq_pad, cnt_pad], axis=1)

    return pl.pallas_call(
        body,
        grid=(_NT // _TB,),
        in_specs=[
            pl.BlockSpec((_TB, _D), lambda i: (i, 0)),
            pl.BlockSpec((_TB,), lambda i: (i,)),
        ],
        out_specs=pl.BlockSpec((_NUM_CLASSES, _AW), lambda i: (0, 0)),
        out_shape=jax.ShapeDtypeStruct((_NUM_CLASSES, _AW), jnp.float32),
        compiler_params=pltpu.CompilerParams(
            dimension_semantics=("arbitrary",)),
    )(z2, c2)


def _tc_finalize(partials3, partial_tc):
    def body(p_ref, t_ref, o_ref):
        a = jnp.sum(p_ref[...], axis=0) + t_ref[...]
        sums = a[:, :_D]
        ssq = jnp.sum(a[:, _D:_D + 16], axis=1)
        cnt = a[:, _D + 16]
        safe = jnp.maximum(cnt, 1.0)
        s2 = jnp.sum(sums * sums, axis=1)
        var_w = (ssq - s2 / safe) / safe
        total = jnp.sum(cnt)
        g = jnp.sum(sums, axis=0) / total
        var_b = jnp.sum(s2 / safe) / total - jnp.sum(g * g)
        loss = (jnp.mean(var_w) + _EPS_WITHIN) / (var_b + _EPS_BETWEEN)
        o_ref[0, 0] = loss

    return pl.pallas_call(
        body,
        out_shape=jax.ShapeDtypeStruct((1, 1), jnp.float32),
        in_specs=[pl.BlockSpec(memory_space=pltpu.VMEM),
                  pl.BlockSpec(memory_space=pltpu.VMEM)],
        out_specs=pl.BlockSpec(memory_space=pltpu.SMEM),
    )(partials3, partial_tc)


@jax.jit
def kernel(z, c):
    z2 = z.reshape(-1, _D)
    c2 = c.reshape(-1)
    partials_sc = _sc_accumulate(z2, c2)
    partial_tc = _tc_partial(z2, c2)
    return _tc_finalize(
        partials_sc.reshape(_NW, _NUM_CLASSES, _AW), partial_tc)[0, 0]


# speedup vs baseline: 1.2777x; 1.0092x over previous
import functools

import jax
import jax.numpy as jnp
from jax import lax
from jax.experimental import pallas as pl
from jax.experimental.pallas import tpu as pltpu
from jax.experimental.pallas import tpu_sc as plsc

_NUM_CLASSES = 64
_EPS_WITHIN = 1e-4
_EPS_BETWEEN = 1e-4

_D = 128
_N = 16 * 4096
_NT = 3 * _N // 4
_SC_ROWS = _N - _NT
_NC = 2
_NW = 16 * _NC
_ROWS_PER_W = _SC_ROWS // _NW
_CHUNK = 256
_NCHUNKS = _ROWS_PER_W // _CHUNK
_TB = 16384
_AW = 160
_ACC_WORDS = _NUM_CLASSES * _AW


def _sc_accumulate(z2, c2):
    mesh = plsc.VectorSubcoreMesh(
        core_axis_name="c", subcore_axis_name="s", num_cores=_NC)

    @functools.partial(
        pl.kernel,
        out_type=jax.ShapeDtypeStruct((_NW, _ACC_WORDS), jnp.float32),
        mesh=mesh,
        scratch_types=[
            pltpu.VMEM((_CHUNK, _D), jnp.float32),
            pltpu.VMEM((_CHUNK, _D), jnp.float32),
            pltpu.VMEM((_ROWS_PER_W,), jnp.int32),
            pltpu.VMEM((_ACC_WORDS,), jnp.float32),
            pltpu.SemaphoreType.DMA,
            pltpu.SemaphoreType.DMA,
        ],
    )
    def body(z_hbm, c_hbm, out_hbm, zbuf0, zbuf1, cbuf, acc, sem0, sem1):
        wid = lax.axis_index("s") * _NC + lax.axis_index("c")
        base = _NT + wid * _ROWS_PER_W
        zbufs = (zbuf0, zbuf1)
        sems = (sem0, sem1)

        def zero_body(i, carry):
            acc[pl.ds(i * 16, 16)] = jnp.zeros((16,), jnp.float32)
            return carry

        copies = [None] * _NCHUNKS

        def start(cc):
            copies[cc] = pltpu.async_copy(
                z_hbm.at[pl.ds(base + cc * _CHUNK, _CHUNK), :],
                zbufs[cc % 2], sems[cc % 2])

        start(0)
        pltpu.sync_copy(c_hbm.at[pl.ds(base, _ROWS_PER_W)], cbuf)
        lax.fori_loop(0, _ACC_WORDS // 16, zero_body, None)

        ones = jnp.ones((16,), jnp.float32)
        zvec = jnp.zeros((16,), jnp.float32)

        for cc in range(_NCHUNKS):
            if cc + 1 < _NCHUNKS:
                start(cc + 1)
            copies[cc].wait()
            zbuf = zbufs[cc % 2]

            def grp_body(g, carry, zbuf=zbuf, cc=cc):
                cvec = cbuf[pl.ds(cc * _CHUNK + g * 16, 16)]
                cls0 = cvec[0]
                uniform = cls0 == cvec[15]

                def fast():
                    off = cls0 * _AW
                    s = [zvec] * (_D // 16)
                    sq = [zvec] * (_D // 16)
                    for t in range(16):
                        r = g * 16 + t
                        for j in range(_D // 16):
                            v = zbuf[r, pl.ds(16 * j, 16)]
                            s[j] = s[j] + v
                            sq[j] = v * v + sq[j]
                    for j in range(_D // 16):
                        plsc.addupdate(acc.at[pl.ds(off + 16 * j, 16)], s[j])
                    sqt = sq[0]
                    for j in range(1, _D // 16):
                        sqt = sqt + sq[j]
                    plsc.addupdate(acc.at[pl.ds(off + _D, 16)], sqt)
                    plsc.addupdate(acc.at[pl.ds(off + _D + 16, 16)], 16.0 * ones)

                def slow():
                    for t in range(16):
                        cls = cvec[t]
                        off = cls * _AW
                        r = g * 16 + t
                        sq = zvec
                        for j in range(_D // 16):
                            v = zbuf[r, pl.ds(16 * j, 16)]
                            plsc.addupdate(acc.at[pl.ds(off + 16 * j, 16)], v)
                            sq = v * v + sq
                        plsc.addupdate(acc.at[pl.ds(off + _D, 16)], sq)
                        plsc.addupdate(acc.at[pl.ds(off + _D + 16, 16)], ones)

                lax.cond(uniform, fast, slow)
                return carry

            lax.fori_loop(0, _CHUNK // 16, grp_body, None)

        pltpu.sync_copy(acc, out_hbm.at[wid])

    return body(z2, c2)


def _tc_partial(z2, c2):
    def body(z_ref, c_ref, o_ref):
        @pl.when(pl.program_id(0) == 0)
        def _():
            o_ref[...] = jnp.zeros_like(o_ref)

        zb = z_ref[...]
        cb = c_ref[...]
        onehot_t = (
            lax.broadcasted_iota(jnp.int32, (_NUM_CLASSES, _TB), 0)
            == cb[None, :]
        ).astype(jnp.float32)
        s1 = jnp.dot(onehot_t, zb, preferred_element_type=jnp.float32)
        sq = jnp.dot(onehot_t, zb * zb,
                     preferred_element_type=jnp.float32)
        ssq = jnp.sum(sq, axis=1, keepdims=True)
        cnt = jnp.sum(onehot_t, axis=1, keepdims=True)
        ssq_pad = jnp.pad(ssq, ((0, 0), (0, 15)))
        cnt_pad = jnp.pad(cnt, ((0, 0), (0, 15)))
        o_ref[...] += jnp.concatenate([s1, ssq_pad, cnt_pad], axis=1)

    return pl.pallas_call(
        body,
        grid=(_NT // _TB,),
        in_specs=[
            pl.BlockSpec((_TB, _D), lambda i: (i, 0)),
            pl.BlockSpec((_TB,), lambda i: (i,)),
        ],
        out_specs=pl.BlockSpec((_NUM_CLASSES, _AW), lambda i: (0, 0)),
        out_shape=jax.ShapeDtypeStruct((_NUM_CLASSES, _AW), jnp.float32),
        compiler_params=pltpu.CompilerParams(
            dimension_semantics=("arbitrary",)),
    )(z2, c2)


def _tc_finalize(partials3, partial_tc):
    def body(p_ref, t_ref, o_ref):
        a = jnp.sum(p_ref[...], axis=0) + t_ref[...]
        sums = a[:, :_D]
        ssq = jnp.sum(a[:, _D:_D + 16], axis=1)
        cnt = a[:, _D + 16]
        safe = jnp.maximum(cnt, 1.0)
        s2 = jnp.sum(sums * sums, axis=1)
        var_w = (ssq - s2 / safe) / safe
        total = jnp.sum(cnt)
        g = jnp.sum(sums, axis=0) / total
        var_b = jnp.sum(s2 / safe) / total - jnp.sum(g * g)
        loss = (jnp.mean(var_w) + _EPS_WITHIN) / (var_b + _EPS_BETWEEN)
        o_ref[0, 0] = loss

    return pl.pallas_call(
        body,
        out_shape=jax.ShapeDtypeStruct((1, 1), jnp.float32),
        in_specs=[pl.BlockSpec(memory_space=pltpu.VMEM),
                  pl.BlockSpec(memory_space=pltpu.VMEM)],
        out_specs=pl.BlockSpec(memory_space=pltpu.SMEM),
    )(partials3, partial_tc)


@jax.jit
def kernel(z, c):
    z2 = z.reshape(-1, _D)
    c2 = c.reshape(-1)
    partials_sc = _sc_accumulate(z2, c2)
    partial_tc = _tc_partial(z2, c2)
    return _tc_finalize(
        partials_sc.reshape(_NW, _NUM_CLASSES, _AW), partial_tc)[0, 0]
